# Initial kernel scaffold; baseline (speedup 1.0000x reference)
import jax
import jax.numpy as jnp
from jax.experimental import pallas as pl


def kernel(x, W_emb, W1, b1, W2, b2, W3, b3):
    raise NotImplementedError("write your pallas kernel here")


# speedup vs baseline: 19.5812x; 19.5812x over previous
import functools

import jax
import jax.numpy as jnp
import numpy as np
from jax import lax
from jax.experimental import pallas as pl
from jax.experimental.pallas import tpu as pltpu
from jax.experimental.pallas import tpu_sc as plsc

_PATTERN_BITS = [16639, 65280, 16711680, 4278190080, 3871, 198415,
                 4345695256, 1108169199648, 283691315109952,
                 72624976668147840, 7357, 135137027, 460551, 1279,
                 134614787, 33693443]

_BATCH = 4096
_FRONT = 64
_MID = 64
_BACK = 32
_NPAT = len(_PATTERN_BITS)
_NSYM = 8
_BAG = _NSYM * _NPAT


def _build_tables():
    mat = np.zeros((64, _NPAT), dtype=np.float32)
    bias = np.zeros((_NPAT,), dtype=np.float32)
    offset = 0
    for p, bits in enumerate(_PATTERN_BITS):
        cells = [i for i in range(64) if (bits >> i) & 1]
        for i, pos in enumerate(cells):
            mat[pos, p] = 3.0 ** i
        bias[p] = float(offset)
        offset += 3 ** len(cells)

    arr = np.arange(64).reshape(1, 1, 8, 8)
    x01 = np.concatenate([arr, np.swapaxes(arr, 2, 3)], axis=1)
    x03 = np.concatenate([x01, np.flip(x01, axis=2)], axis=1)
    x07 = np.concatenate([x03, np.flip(x03, axis=3)], axis=1)
    src = x07.reshape(_NSYM, 64)

    sym_mat = np.zeros((64, _NSYM * _NPAT), dtype=np.float32)
    for k in range(_NSYM):
        sym_mat[src[k, :], 16 * k:16 * (k + 1)] = mat
    sym_bias = np.tile(bias, _NSYM)
    return sym_mat, sym_bias, offset


_SYM_MAT, _SYM_BIAS, _TOTAL_IDX = _build_tables()


def _idx_body(x_ref, sym_ref, bias_ref, o_ref, oob_ref):
    xf = x_ref[...].astype(jnp.float32)
    b = xf[:, :64] + 2.0 * xf[:, 64:]
    s = jnp.dot(b, sym_ref[...], preferred_element_type=jnp.float32)
    s = s + bias_ref[...]
    si = s.astype(jnp.int32)
    oob_ref[...] = jnp.any(si >= _TOTAL_IDX, axis=1, keepdims=True)
    o_ref[...] = jnp.minimum(si, _TOTAL_IDX - 1)


def _compute_indices(x2d):
    return pl.pallas_call(
        _idx_body,
        out_shape=(
            jax.ShapeDtypeStruct((_BATCH, _BAG), jnp.int32),
            jax.ShapeDtypeStruct((_BATCH, 1), jnp.bool_),
        ),
    )(x2d, jnp.asarray(_SYM_MAT), jnp.asarray(_SYM_BIAS).reshape(1, _BAG))


_NC = 2
_NS = 16
_NW = _NC * _NS
_BAGS_PER_W = _BATCH // _NW


def _hsum16(v):
    lane = lax.iota(jnp.int32, 16)
    for sh in (8, 4, 2, 1):
        v = v + v.at[lane ^ sh].get(mode="promise_in_bounds")
    return v


def _vrsqrt(t16):
    i = lax.bitcast_convert_type(t16, jnp.int32)
    i = 0x5F3759DF - lax.shift_right_arithmetic(i, 1)
    y = lax.bitcast_convert_type(i, jnp.float32)
    for _ in range(3):
        y = y * (1.5 - 0.5 * t16 * y * y)
    return y


def _bag_body(idx_hbm, table_hbm, out_hbm, idx_v, rows_v, row_v, sem):
    wid = lax.axis_index("s") * _NC + lax.axis_index("c")

    def bag_fn(i, carry):
        bag = wid * _BAGS_PER_W + i
        pltpu.sync_copy(idx_hbm.at[pl.ds(bag * _BAG, _BAG)], idx_v)
        pltpu.async_copy(table_hbm.at[idx_v], rows_v, sem).wait()

        def row_fn(r, accs):
            a0, a1, a2, a3 = accs
            x0 = rows_v[r, pl.ds(0, 16)]
            x1 = rows_v[r, pl.ds(16, 16)]
            x2 = rows_v[r, pl.ds(32, 16)]
            x3 = rows_v[r, pl.ds(48, 16)]
            t = _hsum16(x0 * x0 + x1 * x1 + x2 * x2 + x3 * x3)
            sv = jnp.where(t > 1.0, _vrsqrt(t), 1.0)
            return (a0 + x0 * sv, a1 + x1 * sv, a2 + x2 * sv, a3 + x3 * sv)

        z = jnp.zeros((16,), jnp.float32)
        a0, a1, a2, a3 = lax.fori_loop(0, _BAG, row_fn, (z, z, z, z))
        row_v[pl.ds(0, 16)] = a0
        row_v[pl.ds(16, 16)] = a1
        row_v[pl.ds(32, 16)] = a2
        row_v[pl.ds(48, 16)] = a3
        pltpu.sync_copy(row_v, out_hbm.at[bag])
        return carry

    lax.fori_loop(0, _BAGS_PER_W, bag_fn, 0)


@functools.cache
def _get_bag_sum():
    return pl.kernel(
        _bag_body,
        out_type=jax.ShapeDtypeStruct((_BATCH, _FRONT), jnp.float32),
        mesh=plsc.VectorSubcoreMesh(core_axis_name="c", subcore_axis_name="s",
                                    num_cores=_NC, num_subcores=_NS),
        scratch_types=[
            pltpu.VMEM((_BAG,), jnp.int32),
            pltpu.VMEM((_BAG, _FRONT), jnp.float32),
            pltpu.VMEM((_FRONT,), jnp.float32),
            pltpu.SemaphoreType.DMA,
        ],
        compiler_params=pltpu.CompilerParams(use_tc_tiling_on_sc=False),
    )


def _mlp_body(m_ref, oob_ref, w1_ref, b1_ref, w2_ref, b2_ref, w3_ref, b3_ref,
              o_ref):
    h = jnp.dot(m_ref[...], w1_ref[...], preferred_element_type=jnp.float32)
    h = jnp.maximum(h + b1_ref[...], 0.0)
    h = jnp.dot(h, w2_ref[...], preferred_element_type=jnp.float32)
    h = jnp.maximum(h + b2_ref[...], 0.0)
    y = jnp.dot(h, w3_ref[...], preferred_element_type=jnp.float32)
    y = y + b3_ref[...]
    o_ref[...] = jnp.where(oob_ref[...], jnp.float32(jnp.nan), y)


def _mlp(m, oob, W1, b1, W2, b2, W3, b3):
    return pl.pallas_call(
        _mlp_body,
        out_shape=jax.ShapeDtypeStruct((_BATCH, 1), jnp.float32),
    )(m, oob, W1, b1.reshape(1, _MID), W2, b2.reshape(1, _BACK), W3,
      b3.reshape(1, 1))


def kernel(x, W_emb, W1, b1, W2, b2, W3, b3):
    x2d = x.reshape(_BATCH, 2 * 64)
    idx, oob = _compute_indices(x2d)
    m = _get_bag_sum()(idx.reshape(-1), W_emb)
    return _mlp(m, oob, W1, b1, W2, b2, W3, b3)


# speedup vs baseline: 21.7755x; 1.1121x over previous
import functools

import jax
import jax.numpy as jnp
import numpy as np
from jax import lax
from jax.experimental import pallas as pl
from jax.experimental.pallas import tpu as pltpu
from jax.experimental.pallas import tpu_sc as plsc

_PATTERN_BITS = [16639, 65280, 16711680, 4278190080, 3871, 198415,
                 4345695256, 1108169199648, 283691315109952,
                 72624976668147840, 7357, 135137027, 460551, 1279,
                 134614787, 33693443]

_BATCH = 4096
_FRONT = 64
_MID = 64
_BACK = 32
_NPAT = len(_PATTERN_BITS)
_NSYM = 8
_BAG = _NSYM * _NPAT


def _build_tables():
    mat = np.zeros((64, _NPAT), dtype=np.float32)
    bias = np.zeros((_NPAT,), dtype=np.float32)
    offset = 0
    for p, bits in enumerate(_PATTERN_BITS):
        cells = [i for i in range(64) if (bits >> i) & 1]
        for i, pos in enumerate(cells):
            mat[pos, p] = 3.0 ** i
        bias[p] = float(offset)
        offset += 3 ** len(cells)

    arr = np.arange(64).reshape(1, 1, 8, 8)
    x01 = np.concatenate([arr, np.swapaxes(arr, 2, 3)], axis=1)
    x03 = np.concatenate([x01, np.flip(x01, axis=2)], axis=1)
    x07 = np.concatenate([x03, np.flip(x03, axis=3)], axis=1)
    src = x07.reshape(_NSYM, 64)

    sym_mat = np.zeros((64, _NSYM * _NPAT), dtype=np.float32)
    for k in range(_NSYM):
        sym_mat[src[k, :], 16 * k:16 * (k + 1)] = mat
    sym_bias = np.tile(bias, _NSYM)
    return sym_mat, sym_bias, offset


_SYM_MAT, _SYM_BIAS, _TOTAL_IDX = _build_tables()


def _idx_body(x_ref, sym_ref, bias_ref, o_ref, oob_ref):
    xf = x_ref[...].astype(jnp.float32)
    b = xf[:, :64] + 2.0 * xf[:, 64:]
    s = jnp.dot(b, sym_ref[...], preferred_element_type=jnp.float32)
    s = s + bias_ref[...]
    si = s.astype(jnp.int32)
    oob_ref[...] = jnp.any(si >= _TOTAL_IDX, axis=1, keepdims=True)
    o_ref[...] = jnp.minimum(si, _TOTAL_IDX - 1)


def _compute_indices(x2d):
    return pl.pallas_call(
        _idx_body,
        out_shape=(
            jax.ShapeDtypeStruct((_BATCH, _BAG), jnp.int32),
            jax.ShapeDtypeStruct((_BATCH, 1), jnp.bool_),
        ),
    )(x2d, jnp.asarray(_SYM_MAT), jnp.asarray(_SYM_BIAS).reshape(1, _BAG))


_NC = 2
_NS = 16
_NW = _NC * _NS
_BAGS_PER_W = _BATCH // _NW


def _hsum16(v):
    lane = lax.iota(jnp.int32, 16)
    for sh in (8, 4, 2, 1):
        v = v + v.at[lane ^ sh].get(mode="promise_in_bounds")
    return v


def _vrsqrt(t16):
    i = lax.bitcast_convert_type(t16, jnp.int32)
    i = 0x5F3759DF - lax.shift_right_arithmetic(i, 1)
    y = lax.bitcast_convert_type(i, jnp.float32)
    for _ in range(3):
        y = y * (1.5 - 0.5 * t16 * y * y)
    return y


_GK = 4
_GROUPS = _BAGS_PER_W // _GK
_PAIRS = _GROUPS // 2


def _bag_body(idx_hbm, table_hbm, out_hbm, idx_v, buf_a, buf_b, out_v,
              sem_a, sem_b):
    wid = lax.axis_index("s") * _NC + lax.axis_index("c")
    base_bag = wid * _BAGS_PER_W
    pltpu.sync_copy(idx_hbm.at[pl.ds(base_bag * _BAG, _BAGS_PER_W * _BAG)],
                    idx_v)

    def fire(g, buf, sem):
        for j in range(_GK):
            pltpu.async_copy(
                table_hbm.at[idx_v.at[pl.ds((g * _GK + j) * _BAG, _BAG)]],
                buf.at[j], sem)

    def drain(buf, sem):
        for j in range(_GK):
            pltpu.make_async_copy(table_hbm.at[idx_v.at[pl.ds(0, _BAG)]],
                                  buf.at[j], sem).wait()

    def compute_group(g, buf):
        for j in range(_GK):
            rows = buf.at[j]

            def row_fn(r, accs):
                a0, a1, a2, a3 = accs
                x0 = rows[r, pl.ds(0, 16)]
                x1 = rows[r, pl.ds(16, 16)]
                x2 = rows[r, pl.ds(32, 16)]
                x3 = rows[r, pl.ds(48, 16)]
                t = _hsum16(x0 * x0 + x1 * x1 + x2 * x2 + x3 * x3)
                sv = jnp.where(t > 1.0, _vrsqrt(t), 1.0)
                return (a0 + x0 * sv, a1 + x1 * sv, a2 + x2 * sv,
                        a3 + x3 * sv)

            z = jnp.zeros((16,), jnp.float32)
            a0, a1, a2, a3 = lax.fori_loop(0, _BAG, row_fn, (z, z, z, z))
            bw = g * _GK + j
            out_v[bw, pl.ds(0, 16)] = a0
            out_v[bw, pl.ds(16, 16)] = a1
            out_v[bw, pl.ds(32, 16)] = a2
            out_v[bw, pl.ds(48, 16)] = a3

    fire(0, buf_a, sem_a)

    def pair_body(t, carry):
        fire(2 * t + 1, buf_b, sem_b)
        drain(buf_a, sem_a)
        compute_group(2 * t, buf_a)

        @pl.when(t < _PAIRS - 1)
        def _():
            fire(2 * t + 2, buf_a, sem_a)

        drain(buf_b, sem_b)
        compute_group(2 * t + 1, buf_b)
        return carry

    lax.fori_loop(0, _PAIRS, pair_body, 0)
    pltpu.sync_copy(out_v, out_hbm.at[pl.ds(base_bag, _BAGS_PER_W)])


@functools.cache
def _get_bag_sum():
    return pl.kernel(
        _bag_body,
        out_type=jax.ShapeDtypeStruct((_BATCH, _FRONT), jnp.float32),
        mesh=plsc.VectorSubcoreMesh(core_axis_name="c", subcore_axis_name="s",
                                    num_cores=_NC, num_subcores=_NS),
        scratch_types=[
            pltpu.VMEM((_BAGS_PER_W * _BAG,), jnp.int32),
            pltpu.VMEM((_GK, _BAG, _FRONT), jnp.float32),
            pltpu.VMEM((_GK, _BAG, _FRONT), jnp.float32),
            pltpu.VMEM((_BAGS_PER_W, _FRONT), jnp.float32),
            pltpu.SemaphoreType.DMA,
            pltpu.SemaphoreType.DMA,
        ],
        compiler_params=pltpu.CompilerParams(use_tc_tiling_on_sc=False),
    )


def _mlp_body(m_ref, oob_ref, w1_ref, b1_ref, w2_ref, b2_ref, w3_ref, b3_ref,
              o_ref):
    h = jnp.dot(m_ref[...], w1_ref[...], preferred_element_type=jnp.float32)
    h = jnp.maximum(h + b1_ref[...], 0.0)
    h = jnp.dot(h, w2_ref[...], preferred_element_type=jnp.float32)
    h = jnp.maximum(h + b2_ref[...], 0.0)
    y = jnp.dot(h, w3_ref[...], preferred_element_type=jnp.float32)
    y = y + b3_ref[...]
    o_ref[...] = jnp.where(oob_ref[...], jnp.float32(jnp.nan), y)


def _mlp(m, oob, W1, b1, W2, b2, W3, b3):
    return pl.pallas_call(
        _mlp_body,
        out_shape=jax.ShapeDtypeStruct((_BATCH, 1), jnp.float32),
    )(m, oob, W1, b1.reshape(1, _MID), W2, b2.reshape(1, _BACK), W3,
      b3.reshape(1, 1))


def kernel(x, W_emb, W1, b1, W2, b2, W3, b3):
    x2d = x.reshape(_BATCH, 2 * 64)
    idx, oob = _compute_indices(x2d)
    m = _get_bag_sum()(idx.reshape(-1), W_emb)
    return _mlp(m, oob, W1, b1, W2, b2, W3, b3)
